# speedup vs baseline: 1.3220x; 1.3220x over previous
import jax
import jax.numpy as jnp
from jax import lax
from jax.experimental import pallas as pl
from jax.experimental.pallas import tpu as pltpu
from jax.experimental.pallas import tpu_sc as plsc

VOCAB = 100000
MODEL_DIM = 128
BATCH = 4
SEQ_LEN = 8192

_NUM_WORKERS = 32
_ROWS_PER_WORKER = BATCH * SEQ_LEN // _NUM_WORKERS
_CHUNK = 512
_GATHER = 128
_N_CHUNKS = _ROWS_PER_WORKER // _CHUNK
_G_PER_CHUNK = _CHUNK // _GATHER


def _frontend_body(x_hbm, emb_hbm, pos_hbm, out_hbm, idx_v, acc_v, sem):
    c = lax.axis_index("c")
    s = lax.axis_index("s")
    wid = s * 2 + c
    pltpu.sync_copy(x_hbm.at[wid], idx_v)
    row0 = wid * _ROWS_PER_WORKER
    pos0 = (wid % (SEQ_LEN // _ROWS_PER_WORKER)) * _ROWS_PER_WORKER
    for h in range(_N_CHUNKS):
        pltpu.sync_copy(pos_hbm.at[pl.ds(pos0 + h * _CHUNK, _CHUNK)], acc_v)
        cps = []
        for j in range(_G_PER_CHUNK):
            cps.append(
                pltpu.async_copy(
                    emb_hbm.at[idx_v.at[h * _G_PER_CHUNK + j]],
                    acc_v.at[pl.ds(j * _GATHER, _GATHER)],
                    sem,
                    add=True,
                )
            )
        for cp in cps:
            cp.wait()
        pltpu.sync_copy(acc_v, out_hbm.at[pl.ds(row0 + h * _CHUNK, _CHUNK)])


@jax.jit
def kernel(x, embed_weight, pos_weight):
    idx = x.reshape(_NUM_WORKERS, _ROWS_PER_WORKER // 128, 128).astype(jnp.int32)
    mesh = plsc.VectorSubcoreMesh(core_axis_name="c", subcore_axis_name="s")
    out = pl.kernel(
        _frontend_body,
        out_type=jax.ShapeDtypeStruct((BATCH * SEQ_LEN, MODEL_DIM), jnp.float32),
        mesh=mesh,
        scratch_types=[
            pltpu.VMEM((_ROWS_PER_WORKER // 128, 128), jnp.int32),
            pltpu.VMEM((_CHUNK, MODEL_DIM), jnp.float32),
            pltpu.SemaphoreType.DMA,
        ],
    )(idx, embed_weight, pos_weight)
    return out.reshape(BATCH, SEQ_LEN, MODEL_DIM)
